# baseline (speedup 1.0000x reference)
import functools

import jax
import jax.numpy as jnp
from jax import lax
from jax.experimental import pallas as pl
from jax.experimental.pallas import tpu as pltpu
from jax.experimental.pallas import tpu_sc as plsc

BATCH = 1024
SEQ = 200
DIM = 128

NUM_IDS = BATCH * SEQ
NC, NS = 2, 16
NW = NC * NS
IDS_PER_W = NUM_IDS // NW
CHUNK = 160
N_CHUNKS = IDS_PER_W // CHUNK
NBUF = 4
assert N_CHUNKS % NBUF == 0


def _sc_gather(W, flat_ids):
    mesh = plsc.VectorSubcoreMesh(core_axis_name="c", subcore_axis_name="s")

    @functools.partial(
        pl.kernel,
        mesh=mesh,
        out_type=jax.ShapeDtypeStruct((NUM_IDS, DIM), W.dtype),
        scratch_types=[
            pltpu.VMEM((IDS_PER_W,), jnp.int32),
            pltpu.VMEM((NBUF, CHUNK, DIM), jnp.float32),
            pltpu.SemaphoreType.DMA((NBUF,)),
            pltpu.SemaphoreType.DMA((NBUF,)),
        ],
    )
    def gather_kernel(w_hbm, ids_hbm, out_hbm, idx_v, rows_v, gsem, osem):
        wid = lax.axis_index("s") * NC + lax.axis_index("c")
        base = wid * IDS_PER_W
        pltpu.sync_copy(ids_hbm.at[pl.ds(base, IDS_PER_W)], idx_v)

        def start_gather(k, b):
            pltpu.make_async_copy(
                w_hbm.at[idx_v.at[pl.ds(k * CHUNK, CHUNK)]],
                rows_v.at[b],
                gsem.at[b],
            ).start()

        def wait_gather(k, b):
            pltpu.make_async_copy(
                w_hbm.at[idx_v.at[pl.ds(k * CHUNK, CHUNK)]],
                rows_v.at[b],
                gsem.at[b],
            ).wait()

        def out_copy(k, b):
            return pltpu.make_async_copy(
                rows_v.at[b],
                out_hbm.at[pl.ds(base + k * CHUNK, CHUNK)],
                osem.at[b],
            )

        start_gather(0, 0)
        start_gather(1, 1)

        @pl.loop(0, N_CHUNKS, step=NBUF)
        def _(c):
            for b in range(NBUF):
                k = c + b
                wait_gather(k, b)
                out_copy(k, b).start()
                bn = (b + 2) % NBUF

                @pl.when(k + 2 < N_CHUNKS)
                def _():
                    @pl.when(k >= 2)
                    def _():
                        out_copy(k - 2, bn).wait()

                    start_gather(k + 2, bn)

        out_copy(N_CHUNKS - 2, (N_CHUNKS - 2) % NBUF).wait()
        out_copy(N_CHUNKS - 1, (N_CHUNKS - 1) % NBUF).wait()

    return gather_kernel(W, flat_ids)


def kernel(input_ids, attention_mask, W):
    flat_ids = input_ids.reshape(NUM_IDS)
    out = _sc_gather(W, flat_ids)
    token_embeddings = out.reshape(BATCH, SEQ, DIM)
    return (input_ids, token_embeddings, attention_mask)


# speedup vs baseline: 1.0097x; 1.0097x over previous
import functools

import jax
import jax.numpy as jnp
from jax import lax
from jax.experimental import pallas as pl
from jax.experimental.pallas import tpu as pltpu
from jax.experimental.pallas import tpu_sc as plsc

BATCH = 1024
SEQ = 200
DIM = 128

NUM_IDS = BATCH * SEQ
NC, NS = 2, 16
NW = NC * NS
IDS_PER_W = NUM_IDS // NW
CHUNK = 128
N_CHUNKS = IDS_PER_W // CHUNK
NBUF = 5
assert N_CHUNKS % NBUF == 0


def _sc_gather(W, flat_ids):
    mesh = plsc.VectorSubcoreMesh(core_axis_name="c", subcore_axis_name="s")

    @functools.partial(
        pl.kernel,
        mesh=mesh,
        out_type=jax.ShapeDtypeStruct((NUM_IDS, DIM), W.dtype),
        scratch_types=[
            pltpu.VMEM((IDS_PER_W,), jnp.int32),
            pltpu.VMEM((NBUF, CHUNK, DIM), jnp.float32),
            pltpu.SemaphoreType.DMA((NBUF,)),
            pltpu.SemaphoreType.DMA((NBUF,)),
        ],
    )
    def gather_kernel(w_hbm, ids_hbm, out_hbm, idx_v, rows_v, gsem, osem):
        wid = lax.axis_index("s") * NC + lax.axis_index("c")
        base = wid * IDS_PER_W
        pltpu.sync_copy(ids_hbm.at[pl.ds(base, IDS_PER_W)], idx_v)

        def start_gather(k, b):
            pltpu.make_async_copy(
                w_hbm.at[idx_v.at[pl.ds(k * CHUNK, CHUNK)]],
                rows_v.at[b],
                gsem.at[b],
            ).start()

        def wait_gather(k, b):
            pltpu.make_async_copy(
                w_hbm.at[idx_v.at[pl.ds(k * CHUNK, CHUNK)]],
                rows_v.at[b],
                gsem.at[b],
            ).wait()

        def out_copy(k, b):
            return pltpu.make_async_copy(
                rows_v.at[b],
                out_hbm.at[pl.ds(base + k * CHUNK, CHUNK)],
                osem.at[b],
            )

        for b in range(NBUF):
            start_gather(b, b)

        @pl.loop(0, N_CHUNKS, step=NBUF)
        def _(c):
            for b in range(NBUF):
                k = c + b
                wait_gather(k, b)
                out_copy(k, b).start()

                @pl.when(k + NBUF < N_CHUNKS)
                def _():
                    out_copy(k, b).wait()
                    start_gather(k + NBUF, b)

        for b in range(NBUF):
            out_copy(N_CHUNKS - NBUF + b, b).wait()

    return gather_kernel(W, flat_ids)


def kernel(input_ids, attention_mask, W):
    flat_ids = input_ids.reshape(NUM_IDS)
    out = _sc_gather(W, flat_ids)
    token_embeddings = out.reshape(BATCH, SEQ, DIM)
    return (input_ids, token_embeddings, attention_mask)


# speedup vs baseline: 1.0171x; 1.0073x over previous
import functools

import jax
import jax.numpy as jnp
from jax import lax
from jax.experimental import pallas as pl
from jax.experimental.pallas import tpu as pltpu
from jax.experimental.pallas import tpu_sc as plsc

BATCH = 1024
SEQ = 200
DIM = 128

NUM_IDS = BATCH * SEQ
NC, NS = 2, 16
NW = NC * NS
IDS_PER_W = NUM_IDS // NW
CHUNK = 80
N_CHUNKS = IDS_PER_W // CHUNK
NBUF = 8
assert N_CHUNKS % NBUF == 0


def _sc_gather(W, flat_ids):
    mesh = plsc.VectorSubcoreMesh(core_axis_name="c", subcore_axis_name="s")

    @functools.partial(
        pl.kernel,
        mesh=mesh,
        out_type=jax.ShapeDtypeStruct((NUM_IDS, DIM), W.dtype),
        scratch_types=[
            pltpu.VMEM((IDS_PER_W,), jnp.int32),
            pltpu.VMEM((NBUF, CHUNK, DIM), jnp.float32),
            pltpu.SemaphoreType.DMA((NBUF,)),
            pltpu.SemaphoreType.DMA((NBUF,)),
        ],
    )
    def gather_kernel(w_hbm, ids_hbm, out_hbm, idx_v, rows_v, gsem, osem):
        wid = lax.axis_index("s") * NC + lax.axis_index("c")
        base = wid * IDS_PER_W
        pltpu.sync_copy(ids_hbm.at[pl.ds(base, IDS_PER_W)], idx_v)

        def start_gather(k, b):
            pltpu.make_async_copy(
                w_hbm.at[idx_v.at[pl.ds(k * CHUNK, CHUNK)]],
                rows_v.at[b],
                gsem.at[b],
            ).start()

        def wait_gather(k, b):
            pltpu.make_async_copy(
                w_hbm.at[idx_v.at[pl.ds(k * CHUNK, CHUNK)]],
                rows_v.at[b],
                gsem.at[b],
            ).wait()

        def out_copy(k, b):
            return pltpu.make_async_copy(
                rows_v.at[b],
                out_hbm.at[pl.ds(base + k * CHUNK, CHUNK)],
                osem.at[b],
            )

        for b in range(NBUF):
            start_gather(b, b)

        @pl.loop(0, N_CHUNKS, step=NBUF)
        def _(c):
            for b in range(NBUF):
                k = c + b
                wait_gather(k, b)
                out_copy(k, b).start()

                @pl.when(k + NBUF < N_CHUNKS)
                def _():
                    out_copy(k, b).wait()
                    start_gather(k + NBUF, b)

        for b in range(NBUF):
            out_copy(N_CHUNKS - NBUF + b, b).wait()

    return gather_kernel(W, flat_ids)


def kernel(input_ids, attention_mask, W):
    flat_ids = input_ids.reshape(NUM_IDS)
    out = _sc_gather(W, flat_ids)
    token_embeddings = out.reshape(BATCH, SEQ, DIM)
    return (input_ids, token_embeddings, attention_mask)
